# baseline (speedup 1.0000x reference)
import functools

import jax
import jax.numpy as jnp
from jax import lax
from jax.experimental import pallas as pl
from jax.experimental.pallas import tpu as pltpu
from jax.experimental.pallas import tpu_sc as plsc

N = 10000
D = 128
E = 320000

C = 40
NTILES = 16
EPT = E // NTILES
CPT = EPT // C
PK = 25
BLOCKS = CPT // PK
IB = PK * C
NB = 9
OG = 6
ZC = 40
NZC = N // ZC
ZPT = -(-NZC // NTILES)


def _segment_sums_sc(x, ps, pd, ns, nd, zeros):
    mesh = plsc.VectorSubcoreMesh(core_axis_name="c", subcore_axis_name="s")

    @functools.partial(
        pl.kernel,
        out_type=(
            jax.ShapeDtypeStruct((N, D), jnp.float32),
            jax.ShapeDtypeStruct((N, D), jnp.float32),
        ),
        mesh=mesh,
        scratch_types=[
            pltpu.VMEM_SHARED((N, D), jnp.float32),
            pltpu.VMEM((2 * IB,), jnp.int32),
            pltpu.VMEM((2 * IB,), jnp.int32),
            pltpu.VMEM((NB, C, D), jnp.float32),
            [pltpu.SemaphoreType.DMA] * NB,
            [pltpu.SemaphoreType.DMA] * NB,
            pltpu.SemaphoreType.DMA,
        ],
    )
    def seg_kernel(x_hbm, ps_hbm, pd_hbm, ns_hbm, nd_hbm, z_hbm,
                   outp_hbm, outn_hbm,
                   acc, src_ring, dst_ring, rows, gsems, ssems, isem):
        cid = lax.axis_index("c")
        sid = lax.axis_index("s")

        def zero_body(t, carry):
            chunk = sid + t * NTILES

            @pl.when(chunk < NZC)
            def _():
                pltpu.sync_copy(z_hbm, acc.at[pl.ds(chunk * ZC, ZC)])

            return carry

        lax.fori_loop(0, ZPT, zero_body, 0)
        plsc.subcore_barrier()

        def run(src_hbm, dst_hbm, out_hbm):
            ebase = sid * EPT

            def load_block(b):
                off = (b % 2) * IB
                pltpu.async_copy(
                    src_hbm.at[pl.ds(ebase + b * IB, IB)],
                    src_ring.at[pl.ds(off, IB)], isem)
                pltpu.async_copy(
                    dst_hbm.at[pl.ds(ebase + b * IB, IB)],
                    dst_ring.at[pl.ds(off, IB)], isem)

            def wait_block():
                for _ in range(2):
                    pltpu.make_async_copy(
                        src_hbm.at[pl.ds(ebase, IB)],
                        src_ring.at[pl.ds(0, IB)], isem).wait()

            def idx_off(t):
                b = t // PK
                return (b % 2) * IB + (t - b * PK) * C

            load_block(0)

            def pipe_body(tt, carry):
                for u in range(NB):
                    t = tt * NB + u
                    j = lax.rem(t, PK)

                    @pl.when(jnp.logical_and(j == 0, t < CPT))
                    def _():
                        wait_block()

                    @pl.when(jnp.logical_and(t >= NB, t < CPT + NB))
                    def _():
                        pltpu.make_async_copy(
                            rows.at[u],
                            acc.at[dst_ring.at[pl.ds(0, C)]],
                            ssems[u]).wait()

                    @pl.when(t < CPT)
                    def _():
                        pltpu.async_copy(
                            x_hbm.at[src_ring.at[pl.ds(idx_off(t), C)]],
                            rows.at[u], gsems[u])

                    v = (u + NB - OG) % NB

                    @pl.when(jnp.logical_and(t >= OG, t < CPT + OG))
                    def _():
                        pltpu.make_async_copy(
                            x_hbm.at[src_ring.at[pl.ds(0, C)]],
                            rows.at[v], gsems[v]).wait()
                        pltpu.async_copy(
                            rows.at[v],
                            acc.at[dst_ring.at[pl.ds(idx_off(t - OG), C)]],
                            ssems[v], add=True)

                    @pl.when(jnp.logical_and(j == 8, t // PK < BLOCKS - 1))
                    def _():
                        load_block(t // PK + 1)

                return carry

            lax.fori_loop(0, (CPT + NB) // NB + 1, pipe_body, 0)
            plsc.subcore_barrier()

            def out_body(t, carry):
                chunk = sid + t * NTILES

                @pl.when(chunk < NZC)
                def _():
                    r0 = chunk * ZC
                    pltpu.sync_copy(acc.at[pl.ds(r0, ZC)], rows.at[0])
                    pltpu.sync_copy(rows.at[0], out_hbm.at[pl.ds(r0, ZC)])

                return carry

            lax.fori_loop(0, ZPT, out_body, 0)

        @pl.when(cid == 0)
        def _():
            run(ps_hbm, pd_hbm, outp_hbm)

        @pl.when(cid == 1)
        def _():
            run(ns_hbm, nd_hbm, outn_hbm)

    return seg_kernel(x, ps, pd, ns, nd, zeros)


def _linear_tc(x, xp, xn, wt, b2):
    BM = 1000

    def mm(x_ref, xp_ref, xn_ref, wt_ref, b_ref, o_ref):
        acc = jnp.dot(x_ref[...], wt_ref[0:D, :],
                      preferred_element_type=jnp.float32)
        acc = acc + jnp.dot(xp_ref[...], wt_ref[D:2 * D, :],
                            preferred_element_type=jnp.float32)
        acc = acc + jnp.dot(xn_ref[...], wt_ref[2 * D:3 * D, :],
                            preferred_element_type=jnp.float32)
        o_ref[...] = acc + b_ref[...]

    return pl.pallas_call(
        mm,
        grid=(N // BM,),
        in_specs=[
            pl.BlockSpec((BM, D), lambda i: (i, 0)),
            pl.BlockSpec((BM, D), lambda i: (i, 0)),
            pl.BlockSpec((BM, D), lambda i: (i, 0)),
            pl.BlockSpec((3 * D, D), lambda i: (0, 0)),
            pl.BlockSpec((1, D), lambda i: (0, 0)),
        ],
        out_specs=pl.BlockSpec((BM, D), lambda i: (i, 0)),
        out_shape=jax.ShapeDtypeStruct((N, D), jnp.float32),
    )(x, xp, xn, wt, b2)


def _split_rows_tc(pe, ne):
    def body(p_ref, n_ref, ps_ref, pd_ref, ns_ref, nd_ref):
        ps_ref[...] = p_ref[0, :]
        pd_ref[...] = p_ref[1, :]
        ns_ref[...] = n_ref[0, :]
        nd_ref[...] = n_ref[1, :]

    return pl.pallas_call(
        body,
        out_shape=[jax.ShapeDtypeStruct((E,), jnp.int32)] * 4,
    )(pe, ne)


def kernel(x, pos_edge_index, neg_edge_index, W, b):
    ps, pd, ns, nd = _split_rows_tc(pos_edge_index.astype(jnp.int32),
                                    neg_edge_index.astype(jnp.int32))
    zeros = jnp.zeros((ZC, D), jnp.float32)
    xp, xn = _segment_sums_sc(x, ps, pd, ns, nd, zeros)
    wt = W.T.reshape(3 * D, D)
    b2 = b.reshape(1, D)
    return _linear_tc(x, xp, xn, wt, b2)


# speedup vs baseline: 1.0712x; 1.0712x over previous
import functools

import jax
import jax.numpy as jnp
from jax import lax
from jax.experimental import pallas as pl
from jax.experimental.pallas import tpu as pltpu
from jax.experimental.pallas import tpu_sc as plsc

N = 10000
D = 128
E = 320000

C = 40
NTILES = 16
EPT = E // NTILES
CPT = EPT // C
PK = 25
BLOCKS = CPT // PK
IB = PK * C
NB = 9
OG = 6
ZC = 40
NZC = N // ZC
ZPT = -(-NZC // NTILES)


def _segment_sums_sc(x, ps, pd, ns, nd, zeros):
    mesh = plsc.VectorSubcoreMesh(core_axis_name="c", subcore_axis_name="s")

    @functools.partial(
        pl.kernel,
        out_type=(
            jax.ShapeDtypeStruct((N, D), jnp.float32),
            jax.ShapeDtypeStruct((N, D), jnp.float32),
        ),
        mesh=mesh,
        scratch_types=[
            pltpu.VMEM_SHARED((N, D), jnp.float32),
            pltpu.VMEM((2 * IB,), jnp.int32),
            pltpu.VMEM((2 * IB,), jnp.int32),
            pltpu.VMEM((NB, C, D), jnp.float32),
            [pltpu.SemaphoreType.DMA] * NB,
            [pltpu.SemaphoreType.DMA] * NB,
            pltpu.SemaphoreType.DMA,
        ],
    )
    def seg_kernel(x_hbm, ps_hbm, pd_hbm, ns_hbm, nd_hbm, z_hbm,
                   outp_hbm, outn_hbm,
                   acc, src_ring, dst_ring, rows, gsems, ssems, isem):
        cid = lax.axis_index("c")
        sid = lax.axis_index("s")

        def zero_body(t, carry):
            chunk = sid + t * NTILES

            @pl.when(chunk < NZC)
            def _():
                pltpu.sync_copy(z_hbm, acc.at[pl.ds(chunk * ZC, ZC)])

            return carry

        lax.fori_loop(0, ZPT, zero_body, 0)
        plsc.subcore_barrier()

        def run(src_hbm, dst_hbm, out_hbm):
            ebase = sid * EPT

            def load_block(b):
                off = (b % 2) * IB
                pltpu.async_copy(
                    src_hbm.at[pl.ds(ebase + b * IB, IB)],
                    src_ring.at[pl.ds(off, IB)], isem)
                pltpu.async_copy(
                    dst_hbm.at[pl.ds(ebase + b * IB, IB)],
                    dst_ring.at[pl.ds(off, IB)], isem)

            def wait_block():
                for _ in range(2):
                    pltpu.make_async_copy(
                        src_hbm.at[pl.ds(ebase, IB)],
                        src_ring.at[pl.ds(0, IB)], isem).wait()

            def idx_off(t):
                b = t // PK
                return (b % 2) * IB + (t - b * PK) * C

            load_block(0)

            def pipe_body(tt, carry):
                for u in range(NB):
                    t = tt * NB + u
                    j = lax.rem(t, PK)

                    @pl.when(jnp.logical_and(j == 0, t < CPT))
                    def _():
                        wait_block()

                    @pl.when(jnp.logical_and(t >= NB, t < CPT + NB))
                    def _():
                        pltpu.make_async_copy(
                            rows.at[u],
                            acc.at[dst_ring.at[pl.ds(0, C)]],
                            ssems[u]).wait()

                    @pl.when(t < CPT)
                    def _():
                        pltpu.async_copy(
                            x_hbm.at[src_ring.at[pl.ds(idx_off(t), C)]],
                            rows.at[u], gsems[u])

                    v = (u + NB - OG) % NB

                    @pl.when(jnp.logical_and(t >= OG, t < CPT + OG))
                    def _():
                        pltpu.make_async_copy(
                            x_hbm.at[src_ring.at[pl.ds(0, C)]],
                            rows.at[v], gsems[v]).wait()
                        pltpu.async_copy(
                            rows.at[v],
                            acc.at[dst_ring.at[pl.ds(idx_off(t - OG), C)]],
                            ssems[v], add=True)

                    @pl.when(jnp.logical_and(j == 8, t // PK < BLOCKS - 1))
                    def _():
                        load_block(t // PK + 1)

                return carry

            lax.fori_loop(0, (CPT + NB) // NB + 1, pipe_body, 0)
            plsc.subcore_barrier()

            def out_body(t, carry):
                chunk = sid + t * NTILES

                @pl.when(chunk < NZC)
                def _():
                    r0 = chunk * ZC
                    pltpu.sync_copy(acc.at[pl.ds(r0, ZC)], rows.at[0])
                    pltpu.sync_copy(rows.at[0], out_hbm.at[pl.ds(r0, ZC)])

                return carry

            lax.fori_loop(0, ZPT, out_body, 0)

        @pl.when(cid == 0)
        def _():
            run(ps_hbm, pd_hbm, outp_hbm)

        @pl.when(cid == 1)
        def _():
            run(ns_hbm, nd_hbm, outn_hbm)

    return seg_kernel(x, ps, pd, ns, nd, zeros)


def _linear_tc(x, xp, xn, wt, b2):
    BM = 1000

    def mm(x_ref, xp_ref, xn_ref, wt_ref, b_ref, o_ref):
        acc = jnp.dot(x_ref[...], wt_ref[0:D, :],
                      preferred_element_type=jnp.float32)
        acc = acc + jnp.dot(xp_ref[...], wt_ref[D:2 * D, :],
                            preferred_element_type=jnp.float32)
        acc = acc + jnp.dot(xn_ref[...], wt_ref[2 * D:3 * D, :],
                            preferred_element_type=jnp.float32)
        o_ref[...] = acc + b_ref[...]

    return pl.pallas_call(
        mm,
        grid=(N // BM,),
        in_specs=[
            pl.BlockSpec((BM, D), lambda i: (i, 0)),
            pl.BlockSpec((BM, D), lambda i: (i, 0)),
            pl.BlockSpec((BM, D), lambda i: (i, 0)),
            pl.BlockSpec((3 * D, D), lambda i: (0, 0)),
            pl.BlockSpec((1, D), lambda i: (0, 0)),
        ],
        out_specs=pl.BlockSpec((BM, D), lambda i: (i, 0)),
        out_shape=jax.ShapeDtypeStruct((N, D), jnp.float32),
    )(x, xp, xn, wt, b2)


def _split_rows_tc(ei):
    def body(e_ref, s_ref, d_ref):
        s_ref[...] = e_ref[0, :]
        d_ref[...] = e_ref[1, :]

    return pl.pallas_call(
        body,
        out_shape=[jax.ShapeDtypeStruct((E,), jnp.int32),
                   jax.ShapeDtypeStruct((E,), jnp.int32)],
    )(ei)


def kernel(x, pos_edge_index, neg_edge_index, W, b):
    ps, pd = _split_rows_tc(pos_edge_index.astype(jnp.int32))
    ns, nd = _split_rows_tc(neg_edge_index.astype(jnp.int32))
    zeros = jnp.zeros((ZC, D), jnp.float32)
    xp, xn = _segment_sums_sc(x, ps, pd, ns, nd, zeros)
    wt = W.T.reshape(3 * D, D)
    b2 = b.reshape(1, D)
    return _linear_tc(x, xp, xn, wt, b2)


# speedup vs baseline: 1.0802x; 1.0085x over previous
import functools

import jax
import jax.numpy as jnp
from jax import lax
from jax.experimental import pallas as pl
from jax.experimental.pallas import tpu as pltpu
from jax.experimental.pallas import tpu_sc as plsc

N = 10000
D = 128
E = 320000

C = 40
NTILES = 16
EPT = E // NTILES
CPT = EPT // C
PK = 20
BLOCKS = CPT // PK
IB = PK * C
NP = 4
ZC = 80
NZC = N // ZC
ZPT = -(-NZC // NTILES)


def _segment_sums_sc(x, ps, pd, ns, nd, zeros):
    mesh = plsc.VectorSubcoreMesh(core_axis_name="c", subcore_axis_name="s")

    @functools.partial(
        pl.kernel,
        out_type=(
            jax.ShapeDtypeStruct((N, D), jnp.float32),
            jax.ShapeDtypeStruct((N, D), jnp.float32),
        ),
        mesh=mesh,
        scratch_types=[
            pltpu.VMEM_SHARED((N, D), jnp.float32),
            pltpu.VMEM((2 * IB,), jnp.int32),
            pltpu.VMEM((2 * IB,), jnp.int32),
            pltpu.VMEM((NP, 2 * C, D), jnp.float32),
            [pltpu.SemaphoreType.DMA] * 8,
            [pltpu.SemaphoreType.DMA] * NP,
            pltpu.SemaphoreType.DMA,
        ],
    )
    def seg_kernel(x_hbm, ps_hbm, pd_hbm, ns_hbm, nd_hbm, z_hbm,
                   outp_hbm, outn_hbm,
                   acc, src_ring, dst_ring, rows, gsems, ssems, isem):
        cid = lax.axis_index("c")
        sid = lax.axis_index("s")

        def zero_body(t, carry):
            chunk = sid + t * NTILES

            @pl.when(chunk < NZC)
            def _():
                pltpu.sync_copy(z_hbm, acc.at[pl.ds(chunk * ZC, ZC)])

            return carry

        lax.fori_loop(0, ZPT, zero_body, 0)
        plsc.subcore_barrier()

        def run(src_hbm, dst_hbm, out_hbm):
            ebase = sid * EPT

            def load_block(b):
                off = (b % 2) * IB
                pltpu.async_copy(
                    src_hbm.at[pl.ds(ebase + b * IB, IB)],
                    src_ring.at[pl.ds(off, IB)], isem)
                pltpu.async_copy(
                    dst_hbm.at[pl.ds(ebase + b * IB, IB)],
                    dst_ring.at[pl.ds(off, IB)], isem)

            def wait_block():
                for _ in range(2):
                    pltpu.make_async_copy(
                        src_hbm.at[pl.ds(ebase, IB)],
                        src_ring.at[pl.ds(0, IB)], isem).wait()

            def idx_off(t):
                b = t // PK
                return (b % 2) * IB + (t - b * PK) * C

            load_block(0)

            def pipe_body(tt, carry):
                for u in range(8):
                    t = tt * 8 + u
                    j = lax.rem(t, PK)

                    if u % 2 == 0:
                        sp = u // 2

                        @pl.when(jnp.logical_and(j == 0, t < CPT))
                        def _():
                            wait_block()

                        @pl.when(jnp.logical_and(t >= 8, t < CPT + 8))
                        def _():
                            pltpu.make_async_copy(
                                rows.at[sp],
                                acc.at[dst_ring.at[pl.ds(0, 2 * C)]],
                                ssems[sp]).wait()

                        @pl.when(t < CPT)
                        def _():
                            pltpu.async_copy(
                                x_hbm.at[
                                    src_ring.at[pl.ds(idx_off(t), C)]],
                                rows.at[sp, pl.ds(0, C)], gsems[u])

                        sv = ((u - 6) // 2) % NP

                        @pl.when(jnp.logical_and(t >= 6, t < CPT + 6))
                        def _():
                            for w in (2, 3):
                                pltpu.make_async_copy(
                                    x_hbm.at[src_ring.at[pl.ds(0, C)]],
                                    rows.at[0, pl.ds(0, C)],
                                    gsems[(u + w) % 8]).wait()
                            pltpu.async_copy(
                                rows.at[sv],
                                acc.at[dst_ring.at[
                                    pl.ds(idx_off(t - 6), 2 * C)]],
                                ssems[sv], add=True)

                        @pl.when(jnp.logical_and(j == 8,
                                                 t // PK < BLOCKS - 1))
                        def _():
                            load_block(t // PK + 1)
                    else:
                        sp = (u - 1) // 2

                        @pl.when(t < CPT)
                        def _():
                            pltpu.async_copy(
                                x_hbm.at[
                                    src_ring.at[pl.ds(idx_off(t), C)]],
                                rows.at[sp, pl.ds(C, C)], gsems[u])

                return carry

            lax.fori_loop(0, (CPT + 8) // 8 + 1, pipe_body, 0)
            plsc.subcore_barrier()

            def out_body(t, carry):
                chunk = sid + t * NTILES

                @pl.when(chunk < NZC)
                def _():
                    r0 = chunk * ZC
                    pltpu.sync_copy(acc.at[pl.ds(r0, ZC)], rows.at[0])
                    pltpu.sync_copy(rows.at[0], out_hbm.at[pl.ds(r0, ZC)])

                return carry

            lax.fori_loop(0, ZPT, out_body, 0)

        @pl.when(cid == 0)
        def _():
            run(ps_hbm, pd_hbm, outp_hbm)

        @pl.when(cid == 1)
        def _():
            run(ns_hbm, nd_hbm, outn_hbm)

    return seg_kernel(x, ps, pd, ns, nd, zeros)


def _linear_tc(x, xp, xn, wt, b2):
    BM = 1000

    def mm(x_ref, xp_ref, xn_ref, wt_ref, b_ref, o_ref):
        acc = jnp.dot(x_ref[...], wt_ref[0:D, :],
                      preferred_element_type=jnp.float32)
        acc = acc + jnp.dot(xp_ref[...], wt_ref[D:2 * D, :],
                            preferred_element_type=jnp.float32)
        acc = acc + jnp.dot(xn_ref[...], wt_ref[2 * D:3 * D, :],
                            preferred_element_type=jnp.float32)
        o_ref[...] = acc + b_ref[...]

    return pl.pallas_call(
        mm,
        grid=(N // BM,),
        in_specs=[
            pl.BlockSpec((BM, D), lambda i: (i, 0)),
            pl.BlockSpec((BM, D), lambda i: (i, 0)),
            pl.BlockSpec((BM, D), lambda i: (i, 0)),
            pl.BlockSpec((3 * D, D), lambda i: (0, 0)),
            pl.BlockSpec((1, D), lambda i: (0, 0)),
        ],
        out_specs=pl.BlockSpec((BM, D), lambda i: (i, 0)),
        out_shape=jax.ShapeDtypeStruct((N, D), jnp.float32),
    )(x, xp, xn, wt, b2)


def _split_rows_tc(ei):
    def body(e_ref, s_ref, d_ref):
        s_ref[...] = e_ref[0, :]
        d_ref[...] = e_ref[1, :]

    return pl.pallas_call(
        body,
        out_shape=[jax.ShapeDtypeStruct((E,), jnp.int32),
                   jax.ShapeDtypeStruct((E,), jnp.int32)],
    )(ei)


def kernel(x, pos_edge_index, neg_edge_index, W, b):
    ps, pd = _split_rows_tc(pos_edge_index.astype(jnp.int32))
    ns, nd = _split_rows_tc(neg_edge_index.astype(jnp.int32))
    zeros = jnp.zeros((ZC, D), jnp.float32)
    xp, xn = _segment_sums_sc(x, ps, pd, ns, nd, zeros)
    wt = W.T.reshape(3 * D, D)
    b2 = b.reshape(1, D)
    return _linear_tc(x, xp, xn, wt, b2)


# speedup vs baseline: 1.0855x; 1.0049x over previous
import functools

import jax
import jax.numpy as jnp
from jax import lax
from jax.experimental import pallas as pl
from jax.experimental.pallas import tpu as pltpu
from jax.experimental.pallas import tpu_sc as plsc

N = 10000
D = 128
E = 320000

C = 40
NTILES = 16
EPT = E // NTILES
CPT = EPT // C
PK = 20
BLOCKS = CPT // PK
IB = PK * C
NP = 4
ZC = 80
NZC = N // ZC
ZPT = -(-NZC // NTILES)


def _segment_sums_sc(x, ps, pd, ns, nd, zeros):
    mesh = plsc.VectorSubcoreMesh(core_axis_name="c", subcore_axis_name="s")

    @functools.partial(
        pl.kernel,
        out_type=(
            jax.ShapeDtypeStruct((N, D), jnp.float32),
            jax.ShapeDtypeStruct((N, D), jnp.float32),
        ),
        mesh=mesh,
        scratch_types=[
            pltpu.VMEM_SHARED((N, D), jnp.float32),
            pltpu.VMEM((2 * IB,), jnp.int32),
            pltpu.VMEM((2 * IB,), jnp.int32),
            pltpu.VMEM((NP, 2 * C, D), jnp.float32),
            [pltpu.SemaphoreType.DMA] * 8,
            [pltpu.SemaphoreType.DMA] * NP,
            pltpu.SemaphoreType.DMA,
        ],
    )
    def seg_kernel(x_hbm, ps_hbm, pd_hbm, ns_hbm, nd_hbm, z_hbm,
                   outp_hbm, outn_hbm,
                   acc, src_ring, dst_ring, rows, gsems, ssems, isem):
        cid = lax.axis_index("c")
        sid = lax.axis_index("s")

        def zero_body(t, carry):
            chunk = sid + t * NTILES

            @pl.when(chunk < NZC)
            def _():
                pltpu.sync_copy(z_hbm, acc.at[pl.ds(chunk * ZC, ZC)])

            return carry

        lax.fori_loop(0, ZPT, zero_body, 0)
        plsc.subcore_barrier()

        def run(src_hbm, dst_hbm, out_hbm):
            ebase = sid * EPT

            def load_block(b):
                off = (b % 2) * IB
                pltpu.async_copy(
                    src_hbm.at[pl.ds(ebase + b * IB, IB)],
                    src_ring.at[pl.ds(off, IB)], isem)
                pltpu.async_copy(
                    dst_hbm.at[pl.ds(ebase + b * IB, IB)],
                    dst_ring.at[pl.ds(off, IB)], isem)

            def wait_block():
                for _ in range(2):
                    pltpu.make_async_copy(
                        src_hbm.at[pl.ds(ebase, IB)],
                        src_ring.at[pl.ds(0, IB)], isem).wait()

            def idx_off(t):
                b = t // PK
                return (b % 2) * IB + (t - b * PK) * C

            load_block(0)

            def pipe_body(tt, carry):
                for u in range(8):
                    t = tt * 8 + u
                    j = lax.rem(t, PK)

                    if u % 2 == 0:
                        sp = u // 2

                        @pl.when(jnp.logical_and(j == 0, t < CPT))
                        def _():
                            wait_block()

                        @pl.when(jnp.logical_and(t >= 8, t < CPT + 8))
                        def _():
                            pltpu.make_async_copy(
                                rows.at[sp],
                                acc.at[dst_ring.at[pl.ds(0, 2 * C)]],
                                ssems[sp]).wait()

                        @pl.when(t < CPT)
                        def _():
                            pltpu.async_copy(
                                x_hbm.at[
                                    src_ring.at[pl.ds(idx_off(t), C)]],
                                rows.at[sp, pl.ds(0, C)], gsems[u])

                        sv = ((u - 6) // 2) % NP

                        @pl.when(jnp.logical_and(t >= 6, t < CPT + 6))
                        def _():
                            for w in (2, 3):
                                pltpu.make_async_copy(
                                    x_hbm.at[src_ring.at[pl.ds(0, C)]],
                                    rows.at[0, pl.ds(0, C)],
                                    gsems[(u + w) % 8]).wait()
                            pltpu.async_copy(
                                rows.at[sv],
                                acc.at[dst_ring.at[
                                    pl.ds(idx_off(t - 6), 2 * C)]],
                                ssems[sv], add=True)

                        @pl.when(jnp.logical_and(j == 8,
                                                 t // PK < BLOCKS - 1))
                        def _():
                            load_block(t // PK + 1)
                    else:
                        sp = (u - 1) // 2

                        @pl.when(t < CPT)
                        def _():
                            pltpu.async_copy(
                                x_hbm.at[
                                    src_ring.at[pl.ds(idx_off(t), C)]],
                                rows.at[sp, pl.ds(C, C)], gsems[u])

                return carry

            lax.fori_loop(0, (CPT + 8) // 8 + 1, pipe_body, 0)
            plsc.subcore_barrier()

            def out_body(t, carry):
                chunk = sid + t * NTILES

                @pl.when(chunk < NZC)
                def _():
                    r0 = chunk * ZC
                    pltpu.sync_copy(acc.at[pl.ds(r0, ZC)],
                                    out_hbm.at[pl.ds(r0, ZC)])

                return carry

            lax.fori_loop(0, ZPT, out_body, 0)

        @pl.when(cid == 0)
        def _():
            run(ps_hbm, pd_hbm, outp_hbm)

        @pl.when(cid == 1)
        def _():
            run(ns_hbm, nd_hbm, outn_hbm)

    return seg_kernel(x, ps, pd, ns, nd, zeros)


def _linear_tc(x, xp, xn, wt, b2):
    BM = 1000

    def mm(x_ref, xp_ref, xn_ref, wt_ref, b_ref, o_ref):
        acc = jnp.dot(x_ref[...], wt_ref[0:D, :],
                      preferred_element_type=jnp.float32)
        acc = acc + jnp.dot(xp_ref[...], wt_ref[D:2 * D, :],
                            preferred_element_type=jnp.float32)
        acc = acc + jnp.dot(xn_ref[...], wt_ref[2 * D:3 * D, :],
                            preferred_element_type=jnp.float32)
        o_ref[...] = acc + b_ref[...]

    return pl.pallas_call(
        mm,
        grid=(N // BM,),
        in_specs=[
            pl.BlockSpec((BM, D), lambda i: (i, 0)),
            pl.BlockSpec((BM, D), lambda i: (i, 0)),
            pl.BlockSpec((BM, D), lambda i: (i, 0)),
            pl.BlockSpec((3 * D, D), lambda i: (0, 0)),
            pl.BlockSpec((1, D), lambda i: (0, 0)),
        ],
        out_specs=pl.BlockSpec((BM, D), lambda i: (i, 0)),
        out_shape=jax.ShapeDtypeStruct((N, D), jnp.float32),
    )(x, xp, xn, wt, b2)


def _split_rows_tc(ei):
    def body(e_ref, s_ref, d_ref):
        s_ref[...] = e_ref[0, :]
        d_ref[...] = e_ref[1, :]

    return pl.pallas_call(
        body,
        out_shape=[jax.ShapeDtypeStruct((E,), jnp.int32),
                   jax.ShapeDtypeStruct((E,), jnp.int32)],
    )(ei)


def kernel(x, pos_edge_index, neg_edge_index, W, b):
    ps, pd = _split_rows_tc(pos_edge_index.astype(jnp.int32))
    ns, nd = _split_rows_tc(neg_edge_index.astype(jnp.int32))
    zeros = jnp.zeros((ZC, D), jnp.float32)
    xp, xn = _segment_sums_sc(x, ps, pd, ns, nd, zeros)
    wt = W.T.reshape(3 * D, D)
    b2 = b.reshape(1, D)
    return _linear_tc(x, xp, xn, wt, b2)


# speedup vs baseline: 1.1011x; 1.0144x over previous
import functools

import jax
import jax.numpy as jnp
from jax import lax
from jax.experimental import pallas as pl
from jax.experimental.pallas import tpu as pltpu
from jax.experimental.pallas import tpu_sc as plsc

N = 10000
D = 128
E = 320000

C = 40
NTILES = 16
EPT = E // NTILES
CPT = EPT // C
PK = 20
BLOCKS = CPT // PK
IB = PK * C
NP = 4
ZC = 80
NZC = N // ZC
ZPT = -(-NZC // NTILES)


def _segment_sums_sc(x, ps, pd, ns, nd, zeros):
    mesh = plsc.VectorSubcoreMesh(core_axis_name="c", subcore_axis_name="s")

    @functools.partial(
        pl.kernel,
        out_type=(
            jax.ShapeDtypeStruct((N, D), jnp.float32),
            jax.ShapeDtypeStruct((N, D), jnp.float32),
        ),
        mesh=mesh,
        scratch_types=[
            pltpu.VMEM_SHARED((N, D), jnp.float32),
            pltpu.VMEM((2 * IB,), jnp.int32),
            pltpu.VMEM((2 * IB,), jnp.int32),
            pltpu.VMEM((NP, 2 * C, D), jnp.float32),
            [pltpu.SemaphoreType.DMA] * 8,
            [pltpu.SemaphoreType.DMA] * NP,
            pltpu.SemaphoreType.DMA,
        ],
    )
    def seg_kernel(x_hbm, ps_hbm, pd_hbm, ns_hbm, nd_hbm, z_hbm,
                   outp_hbm, outn_hbm,
                   acc, src_ring, dst_ring, rows, gsems, ssems, isem):
        cid = lax.axis_index("c")
        sid = lax.axis_index("s")

        def zero_body(t, carry):
            chunk = sid + t * NTILES

            @pl.when(chunk < NZC)
            def _():
                pltpu.sync_copy(z_hbm, acc.at[pl.ds(chunk * ZC, ZC)])

            return carry

        lax.fori_loop(0, ZPT, zero_body, 0)
        plsc.subcore_barrier()

        def run(src_hbm, dst_hbm, out_hbm):
            ebase = sid * EPT

            def load_block(b):
                off = (b % 2) * IB
                pltpu.async_copy(
                    src_hbm.at[pl.ds(ebase + b * IB, IB)],
                    src_ring.at[pl.ds(off, IB)], isem)
                pltpu.async_copy(
                    dst_hbm.at[pl.ds(ebase + b * IB, IB)],
                    dst_ring.at[pl.ds(off, IB)], isem)

            def wait_block():
                for _ in range(2):
                    pltpu.make_async_copy(
                        src_hbm.at[pl.ds(ebase, IB)],
                        src_ring.at[pl.ds(0, IB)], isem).wait()

            def idx_off(t):
                b = t // PK
                return (b % 2) * IB + (t - b * PK) * C

            load_block(0)

            def pipe_body(tt, carry):
                for u in range(8):
                    t = tt * 8 + u
                    j = lax.rem(t, PK)

                    if u % 2 == 0:
                        sp = u // 2

                        @pl.when(jnp.logical_and(j == 0, t < CPT))
                        def _():
                            wait_block()

                        @pl.when(jnp.logical_and(t >= 8, t < CPT + 8))
                        def _():
                            pltpu.make_async_copy(
                                rows.at[sp],
                                acc.at[dst_ring.at[pl.ds(0, 2 * C)]],
                                ssems[sp]).wait()

                        @pl.when(t < CPT)
                        def _():
                            pltpu.async_copy(
                                x_hbm.at[
                                    src_ring.at[pl.ds(idx_off(t), C)]],
                                rows.at[sp, pl.ds(0, C)], gsems[u])

                        sv = ((u - 6) // 2) % NP

                        @pl.when(jnp.logical_and(t >= 6, t < CPT + 6))
                        def _():
                            for w in (2, 3):
                                pltpu.make_async_copy(
                                    x_hbm.at[src_ring.at[pl.ds(0, C)]],
                                    rows.at[0, pl.ds(0, C)],
                                    gsems[(u + w) % 8]).wait()
                            pltpu.async_copy(
                                rows.at[sv],
                                acc.at[dst_ring.at[
                                    pl.ds(idx_off(t - 6), 2 * C)]],
                                ssems[sv], add=True)

                        @pl.when(jnp.logical_and(j == 8,
                                                 t // PK < BLOCKS - 1))
                        def _():
                            load_block(t // PK + 1)
                    else:
                        sp = (u - 1) // 2

                        @pl.when(t < CPT)
                        def _():
                            pltpu.async_copy(
                                x_hbm.at[
                                    src_ring.at[pl.ds(idx_off(t), C)]],
                                rows.at[sp, pl.ds(C, C)], gsems[u])

                return carry

            lax.fori_loop(0, (CPT + 8) // 8 + 1, pipe_body, 0)
            plsc.subcore_barrier()

            def out_body(t, carry):
                chunk = sid + t * NTILES

                @pl.when(chunk < NZC)
                def _():
                    r0 = chunk * ZC
                    pltpu.sync_copy(acc.at[pl.ds(r0, ZC)],
                                    out_hbm.at[pl.ds(r0, ZC)])

                return carry

            lax.fori_loop(0, ZPT, out_body, 0)

        @pl.when(cid == 0)
        def _():
            run(ps_hbm, pd_hbm, outp_hbm)

        @pl.when(cid == 1)
        def _():
            run(ns_hbm, nd_hbm, outn_hbm)

    return seg_kernel(x, ps, pd, ns, nd, zeros)


def _linear_tc(x, xp, xn, wt, b2):
    BM = 2000

    def mm(x_ref, xp_ref, xn_ref, wt_ref, b_ref, o_ref):
        acc = jnp.dot(x_ref[...], wt_ref[0:D, :],
                      preferred_element_type=jnp.float32)
        acc = acc + jnp.dot(xp_ref[...], wt_ref[D:2 * D, :],
                            preferred_element_type=jnp.float32)
        acc = acc + jnp.dot(xn_ref[...], wt_ref[2 * D:3 * D, :],
                            preferred_element_type=jnp.float32)
        o_ref[...] = acc + b_ref[...]

    return pl.pallas_call(
        mm,
        grid=(N // BM,),
        in_specs=[
            pl.BlockSpec((BM, D), lambda i: (i, 0)),
            pl.BlockSpec((BM, D), lambda i: (i, 0)),
            pl.BlockSpec((BM, D), lambda i: (i, 0)),
            pl.BlockSpec((3 * D, D), lambda i: (0, 0)),
            pl.BlockSpec((1, D), lambda i: (0, 0)),
        ],
        out_specs=pl.BlockSpec((BM, D), lambda i: (i, 0)),
        out_shape=jax.ShapeDtypeStruct((N, D), jnp.float32),
    )(x, xp, xn, wt, b2)


def _split_rows_tc(ei):
    def body(e_ref, s_ref, d_ref):
        s_ref[...] = e_ref[0, :]
        d_ref[...] = e_ref[1, :]

    return pl.pallas_call(
        body,
        out_shape=[jax.ShapeDtypeStruct((E,), jnp.int32),
                   jax.ShapeDtypeStruct((E,), jnp.int32)],
    )(ei)


def kernel(x, pos_edge_index, neg_edge_index, W, b):
    ps, pd = _split_rows_tc(pos_edge_index.astype(jnp.int32))
    ns, nd = _split_rows_tc(neg_edge_index.astype(jnp.int32))
    zeros = jnp.zeros((ZC, D), jnp.float32)
    xp, xn = _segment_sums_sc(x, ps, pd, ns, nd, zeros)
    wt = W.T.reshape(3 * D, D)
    b2 = b.reshape(1, D)
    return _linear_tc(x, xp, xn, wt, b2)


# speedup vs baseline: 1.1015x; 1.0003x over previous
import functools

import jax
import jax.numpy as jnp
from jax import lax
from jax.experimental import pallas as pl
from jax.experimental.pallas import tpu as pltpu
from jax.experimental.pallas import tpu_sc as plsc

N = 10000
D = 128
E = 320000

C = 40
NTILES = 16
EPT = E // NTILES
CPT = EPT // C
PK = 20
BLOCKS = CPT // PK
IB = PK * C
NP = 4
ZC = 80
NZC = N // ZC
ZPT = -(-NZC // NTILES)


def _segment_sums_sc(x, ps, pd, ns, nd, zeros):
    mesh = plsc.VectorSubcoreMesh(core_axis_name="c", subcore_axis_name="s")

    @functools.partial(
        pl.kernel,
        out_type=(
            jax.ShapeDtypeStruct((N, D), jnp.float32),
            jax.ShapeDtypeStruct((N, D), jnp.float32),
        ),
        mesh=mesh,
        scratch_types=[
            pltpu.VMEM_SHARED((N, D), jnp.float32),
            pltpu.VMEM((2 * IB,), jnp.int32),
            pltpu.VMEM((2 * IB,), jnp.int32),
            pltpu.VMEM((NP, 2 * C, D), jnp.float32),
            [pltpu.SemaphoreType.DMA] * 8,
            [pltpu.SemaphoreType.DMA] * NP,
            pltpu.SemaphoreType.DMA,
        ],
    )
    def seg_kernel(x_hbm, ps_hbm, pd_hbm, ns_hbm, nd_hbm, z_hbm,
                   outp_hbm, outn_hbm,
                   acc, src_ring, dst_ring, rows, gsems, ssems, isem):
        cid = lax.axis_index("c")
        sid = lax.axis_index("s")

        def zero_body(t, carry):
            chunk = sid + t * NTILES

            @pl.when(chunk < NZC)
            def _():
                pltpu.async_copy(z_hbm, acc.at[pl.ds(chunk * ZC, ZC)], isem)

            return carry

        lax.fori_loop(0, ZPT, zero_body, 0)

        def zero_drain(t, carry):
            @pl.when(sid + t * NTILES < NZC)
            def _():
                pltpu.make_async_copy(
                    z_hbm, acc.at[pl.ds(0, ZC)], isem).wait()

            return carry

        lax.fori_loop(0, ZPT, zero_drain, 0)
        plsc.subcore_barrier()

        def run(src_hbm, dst_hbm, out_hbm):
            ebase = sid * EPT

            def load_block(b):
                off = (b % 2) * IB
                pltpu.async_copy(
                    src_hbm.at[pl.ds(ebase + b * IB, IB)],
                    src_ring.at[pl.ds(off, IB)], isem)
                pltpu.async_copy(
                    dst_hbm.at[pl.ds(ebase + b * IB, IB)],
                    dst_ring.at[pl.ds(off, IB)], isem)

            def wait_block():
                for _ in range(2):
                    pltpu.make_async_copy(
                        src_hbm.at[pl.ds(ebase, IB)],
                        src_ring.at[pl.ds(0, IB)], isem).wait()

            def idx_off(t):
                b = t // PK
                return (b % 2) * IB + (t - b * PK) * C

            load_block(0)

            def pipe_body(tt, carry):
                for u in range(8):
                    t = tt * 8 + u
                    j = lax.rem(t, PK)

                    if u % 2 == 0:
                        sp = u // 2

                        @pl.when(jnp.logical_and(j == 0, t < CPT))
                        def _():
                            wait_block()

                        @pl.when(jnp.logical_and(t >= 8, t < CPT + 8))
                        def _():
                            pltpu.make_async_copy(
                                rows.at[sp],
                                acc.at[dst_ring.at[pl.ds(0, 2 * C)]],
                                ssems[sp]).wait()

                        @pl.when(t < CPT)
                        def _():
                            pltpu.async_copy(
                                x_hbm.at[
                                    src_ring.at[pl.ds(idx_off(t), C)]],
                                rows.at[sp, pl.ds(0, C)], gsems[u])

                        sv = ((u - 6) // 2) % NP

                        @pl.when(jnp.logical_and(t >= 6, t < CPT + 6))
                        def _():
                            for w in (2, 3):
                                pltpu.make_async_copy(
                                    x_hbm.at[src_ring.at[pl.ds(0, C)]],
                                    rows.at[0, pl.ds(0, C)],
                                    gsems[(u + w) % 8]).wait()
                            pltpu.async_copy(
                                rows.at[sv],
                                acc.at[dst_ring.at[
                                    pl.ds(idx_off(t - 6), 2 * C)]],
                                ssems[sv], add=True)

                        @pl.when(jnp.logical_and(j == 8,
                                                 t // PK < BLOCKS - 1))
                        def _():
                            load_block(t // PK + 1)
                    else:
                        sp = (u - 1) // 2

                        @pl.when(t < CPT)
                        def _():
                            pltpu.async_copy(
                                x_hbm.at[
                                    src_ring.at[pl.ds(idx_off(t), C)]],
                                rows.at[sp, pl.ds(C, C)], gsems[u])

                return carry

            lax.fori_loop(0, (CPT + 8) // 8 + 1, pipe_body, 0)
            plsc.subcore_barrier()

            def out_body(t, carry):
                chunk = sid + t * NTILES

                @pl.when(chunk < NZC)
                def _():
                    r0 = chunk * ZC
                    pltpu.sync_copy(acc.at[pl.ds(r0, ZC)],
                                    out_hbm.at[pl.ds(r0, ZC)])

                return carry

            lax.fori_loop(0, ZPT, out_body, 0)

        @pl.when(cid == 0)
        def _():
            run(ps_hbm, pd_hbm, outp_hbm)

        @pl.when(cid == 1)
        def _():
            run(ns_hbm, nd_hbm, outn_hbm)

    return seg_kernel(x, ps, pd, ns, nd, zeros)


def _linear_tc(x, xp, xn, wt, b2):
    BM = 2000

    def mm(x_ref, xp_ref, xn_ref, wt_ref, b_ref, o_ref):
        acc = jnp.dot(x_ref[...], wt_ref[0:D, :],
                      preferred_element_type=jnp.float32)
        acc = acc + jnp.dot(xp_ref[...], wt_ref[D:2 * D, :],
                            preferred_element_type=jnp.float32)
        acc = acc + jnp.dot(xn_ref[...], wt_ref[2 * D:3 * D, :],
                            preferred_element_type=jnp.float32)
        o_ref[...] = acc + b_ref[...]

    return pl.pallas_call(
        mm,
        grid=(N // BM,),
        in_specs=[
            pl.BlockSpec((BM, D), lambda i: (i, 0)),
            pl.BlockSpec((BM, D), lambda i: (i, 0)),
            pl.BlockSpec((BM, D), lambda i: (i, 0)),
            pl.BlockSpec((3 * D, D), lambda i: (0, 0)),
            pl.BlockSpec((1, D), lambda i: (0, 0)),
        ],
        out_specs=pl.BlockSpec((BM, D), lambda i: (i, 0)),
        out_shape=jax.ShapeDtypeStruct((N, D), jnp.float32),
    )(x, xp, xn, wt, b2)


def _split_rows_tc(ei):
    def body(e_ref, s_ref, d_ref):
        s_ref[...] = e_ref[0, :]
        d_ref[...] = e_ref[1, :]

    return pl.pallas_call(
        body,
        out_shape=[jax.ShapeDtypeStruct((E,), jnp.int32),
                   jax.ShapeDtypeStruct((E,), jnp.int32)],
    )(ei)


def kernel(x, pos_edge_index, neg_edge_index, W, b):
    ps, pd = _split_rows_tc(pos_edge_index.astype(jnp.int32))
    ns, nd = _split_rows_tc(neg_edge_index.astype(jnp.int32))
    zeros = jnp.zeros((ZC, D), jnp.float32)
    xp, xn = _segment_sums_sc(x, ps, pd, ns, nd, zeros)
    wt = W.T.reshape(3 * D, D)
    b2 = b.reshape(1, D)
    return _linear_tc(x, xp, xn, wt, b2)
